# baseline (speedup 1.0000x reference)
import functools

import jax
import jax.numpy as jnp
from jax import lax
from jax.experimental import pallas as pl
from jax.experimental.pallas import tpu as pltpu
from jax.experimental.pallas import tpu_sc as plsc

VOCAB = 1000000
DIM = 32
SEQ = 50
BATCH = 16384

_info = plsc.get_sparse_core_info()
_NC, _NS = _info.num_cores, _info.num_subcores
_NW = _NC * _NS
_IPW = BATCH // _NW
_ICHUNK = 16
_NCHUNK = _IPW // _ICHUNK

_TR = 800
_NTC = VOCAB // _TR
_TPT = (_NTC + _NS - 1) // _NS


@functools.partial(
    pl.kernel,
    mesh=plsc.VectorSubcoreMesh(core_axis_name="c", subcore_axis_name="s"),
    out_type=(
        jax.ShapeDtypeStruct((BATCH, SEQ, DIM), jnp.float32),
        jax.ShapeDtypeStruct((VOCAB, DIM), jnp.float32),
    ),
    scratch_types=[
        pltpu.VMEM((DIM, _TR), jnp.float32),
        pltpu.VMEM((_TR, DIM), jnp.float32),
        pltpu.VMEM((SEQ, _ICHUNK), jnp.int32),
        pltpu.VMEM((_ICHUNK, SEQ), jnp.int32),
        pltpu.VMEM((_ICHUNK, SEQ), jnp.int32),
        pltpu.VMEM((_ICHUNK, SEQ, DIM), jnp.float32),
        pltpu.VMEM((_ICHUNK, SEQ, DIM), jnp.float32),
        pltpu.SemaphoreType.DMA,
        pltpu.SemaphoreType.DMA,
        pltpu.SemaphoreType.DMA,
        pltpu.SemaphoreType.DMA,
    ],
    compiler_params=pltpu.CompilerParams(use_tc_tiling_on_sc=False,
                                         needs_layout_passes=False),
)
def _embed_kernel(xT, tableT, out_hbm, tprime, A, B, xb, idx0, idx1,
                  rows0, rows1, gsem0, gsem1, ssem0, ssem1):
    sid = lax.axis_index("s")
    wid = sid * _NC + lax.axis_index("c")
    iota16 = lax.iota(jnp.int32, 16)

    def tr_chunk(t, carry):
        c = sid + t * _NS

        @pl.when(c < _NTC)
        def _():
            r0 = c * _TR
            pltpu.sync_copy(tableT.at[:, pl.ds(r0, _TR)], A)

            def tr_d(d, carry2):
                dsplat = jnp.full((16,), d, jnp.int32)
                for rv in range(_TR // 16):
                    v = A[d, pl.ds(rv * 16, 16)]
                    plsc.store_scatter(B, [rv * 16 + iota16, dsplat], v)
                return carry2

            lax.fori_loop(0, DIM, tr_d, 0)
            pltpu.sync_copy(B, tprime.at[pl.ds(r0, _TR)])

        return carry

    lax.fori_loop(0, _TPT, tr_chunk, 0)
    plsc.subcore_barrier()

    ibase = wid * _IPW
    idxs = (idx0, idx1)
    rows = (rows0, rows1)
    gsems = (gsem0, gsem1)
    ssems = (ssem0, ssem1)

    def fire_chunk(cc, b):
        i0 = ibase + cc * _ICHUNK
        pltpu.sync_copy(xT.at[:, pl.ds(i0, _ICHUNK)], xb)
        idxb = idxs[b]

        def tp_j(j, carry):
            jsplat = jnp.full((16,), j, jnp.int32)
            v = xb[j, pl.ds(0, 16)]
            plsc.store_scatter(idxb, [iota16, jsplat], v)
            return carry

        lax.fori_loop(0, SEQ, tp_j, 0)

        def gather_one(k, carry):
            pltpu.async_copy(tprime.at[idxb.at[k]], rows[b].at[k], gsems[b])
            return carry

        lax.fori_loop(0, _ICHUNK, gather_one, 0)

    def drain_chunk(cc, b):
        i0 = ibase + cc * _ICHUNK
        pltpu.make_async_copy(out_hbm.at[pl.ds(i0, _ICHUNK)], rows[b],
                              gsems[b]).wait()

    stores = [None] * _NCHUNK
    for cc in range(_NCHUNK):
        b = cc % 2
        if cc >= 2:
            stores[cc - 2].wait()
        fire_chunk(cc, b)
        if cc >= 1:
            p = 1 - b
            drain_chunk(cc - 1, p)
            stores[cc - 1] = pltpu.async_copy(
                rows[p],
                out_hbm.at[pl.ds(ibase + (cc - 1) * _ICHUNK, _ICHUNK)],
                ssems[p])
    last = _NCHUNK - 1
    drain_chunk(last, last % 2)
    stores[last] = pltpu.async_copy(
        rows[last % 2], out_hbm.at[pl.ds(ibase + last * _ICHUNK, _ICHUNK)],
        ssems[last % 2])
    stores[last - 1].wait()
    stores[last].wait()


def kernel(x, table):
    xT = jnp.transpose(x.astype(jnp.int32))
    tT = jnp.transpose(table)
    out, _ = _embed_kernel(xT, tT)
    return out


# speedup vs baseline: 1.1245x; 1.1245x over previous
import functools

import jax
import jax.numpy as jnp
from jax import lax
from jax.experimental import pallas as pl
from jax.experimental.pallas import tpu as pltpu
from jax.experimental.pallas import tpu_sc as plsc

VOCAB = 1000000
DIM = 32
SEQ = 50
BATCH = 16384

_info = plsc.get_sparse_core_info()
_NC, _NS = _info.num_cores, _info.num_subcores
_NW = _NC * _NS
_IPW = BATCH // _NW
_ICHUNK = 16
_NCHUNK = _IPW // _ICHUNK

_TR = 400
_NTC = VOCAB // _TR
_TPT = (_NTC + _NS - 1) // _NS
_TPT2 = (_TPT + 1) // 2


@functools.partial(
    pl.kernel,
    mesh=plsc.VectorSubcoreMesh(core_axis_name="c", subcore_axis_name="s"),
    out_type=(
        jax.ShapeDtypeStruct((BATCH, SEQ, DIM), jnp.float32),
        jax.ShapeDtypeStruct((VOCAB, DIM), jnp.float32),
    ),
    scratch_types=[
        pltpu.VMEM((DIM * _TR,), jnp.float32),
        pltpu.VMEM((DIM * _TR,), jnp.float32),
        pltpu.VMEM((_TR, DIM), jnp.float32),
        pltpu.VMEM((_TR, DIM), jnp.float32),
        pltpu.VMEM((SEQ, _ICHUNK), jnp.int32),
        pltpu.VMEM((_ICHUNK, SEQ), jnp.int32),
        pltpu.VMEM((_ICHUNK, SEQ), jnp.int32),
        pltpu.VMEM((_ICHUNK, SEQ, DIM), jnp.float32),
        pltpu.VMEM((_ICHUNK, SEQ, DIM), jnp.float32),
        pltpu.SemaphoreType.DMA,
        pltpu.SemaphoreType.DMA,
        pltpu.SemaphoreType.DMA,
        pltpu.SemaphoreType.DMA,
        pltpu.SemaphoreType.DMA,
        pltpu.SemaphoreType.DMA,
        pltpu.SemaphoreType.DMA,
        pltpu.SemaphoreType.DMA,
    ],
    compiler_params=pltpu.CompilerParams(use_tc_tiling_on_sc=False,
                                         needs_layout_passes=False,
                                         disable_bounds_checks=True),
)
def _embed_kernel(xT, tableT, out_hbm, tprime, A0, A1, B0, B1, xb,
                  idx0, idx1, rows0, rows1,
                  asem0, asem1, bsem0, bsem1, gsem0, gsem1, ssem0, ssem1):
    sid = lax.axis_index("s")
    wid = sid * _NC + lax.axis_index("c")
    iota16 = lax.iota(jnp.int32, 16)
    iotaT = (iota16 * _TR, iota16 * _TR + 16 * _TR)

    As = (A0, A1)
    Bs = (B0, B1)
    asems = (asem0, asem1)
    bsems = (bsem0, bsem1)

    def cond(t):
        return (sid + t * _NS) < _NTC

    def chunk_r0(t):
        return (sid + t * _NS) * _TR

    def aload(t, b):
        @pl.when(cond(t))
        def _():
            r0 = chunk_r0(t)
            for d in range(DIM):
                pltpu.async_copy(tableT.at[d, pl.ds(r0, _TR)],
                                 As[b].at[pl.ds(d * _TR, _TR)], asems[b])

    def await_a(t, b):
        @pl.when(cond(t))
        def _():
            r0 = chunk_r0(t)
            for d in range(DIM):
                pltpu.make_async_copy(
                    tableT.at[d, pl.ds(r0, _TR)],
                    As[b].at[pl.ds(d * _TR, _TR)], asems[b]).wait()

    def transpose_chunk(b):
        A1d = As[b]
        B2 = Bs[b]

        def body(r8, carry):
            for rr in range(8):
                r = r8 * 8 + rr
                for half in range(2):
                    idx = iotaT[half] + r
                    v = plsc.load_gather(A1d, [idx])
                    B2[r, pl.ds(half * 16, 16)] = v
            return carry

        lax.fori_loop(0, _TR // 8, body, 0)

    def bstore(t, b):
        @pl.when(cond(t))
        def _():
            pltpu.async_copy(Bs[b], tprime.at[pl.ds(chunk_r0(t), _TR)],
                             bsems[b])

    def bwait(pred, t, b):
        @pl.when(pred)
        def _():
            pltpu.make_async_copy(Bs[b],
                                  tprime.at[pl.ds(chunk_r0(t), _TR)],
                                  bsems[b]).wait()

    aload(0, 0)

    def step(t2, carry):
        for bb in range(2):
            t = t2 * 2 + bb
            await_a(t, bb)
            aload_t = t + 1
            aload(aload_t, 1 - bb)
            bwait(jnp.logical_and(t2 >= 1, cond(t)), t - 2, bb)

            @pl.when(cond(t))
            def _():
                transpose_chunk(bb)

            bstore(t, bb)
        return carry

    lax.fori_loop(0, _TPT2, step, 0)
    for tt in range(_TPT - 5, _TPT + 1):
        bwait(jnp.logical_and(cond(tt), jnp.logical_not(cond(tt + 2))),
              tt, tt % 2)
    plsc.subcore_barrier()

    ibase = wid * _IPW
    idxs = (idx0, idx1)
    rows = (rows0, rows1)
    gsems = (gsem0, gsem1)
    ssems = (ssem0, ssem1)

    def fire_chunk(cc, b):
        i0 = ibase + cc * _ICHUNK
        pltpu.sync_copy(xT.at[:, pl.ds(i0, _ICHUNK)], xb)
        idxb = idxs[b]

        def tp_j(j, carry):
            jsplat = jnp.full((16,), j, jnp.int32)
            v = xb[j, pl.ds(0, 16)]
            plsc.store_scatter(idxb, [iota16, jsplat], v)
            return carry

        lax.fori_loop(0, SEQ, tp_j, 0)

        def gather_one(k, carry):
            pltpu.async_copy(tprime.at[idxb.at[k]], rows[b].at[k], gsems[b])
            return carry

        lax.fori_loop(0, _ICHUNK, gather_one, 0)

    def drain_chunk(cc, b):
        i0 = ibase + cc * _ICHUNK
        pltpu.make_async_copy(out_hbm.at[pl.ds(i0, _ICHUNK)], rows[b],
                              gsems[b]).wait()

    stores = [None] * _NCHUNK
    for cc in range(_NCHUNK):
        b = cc % 2
        if cc >= 2:
            stores[cc - 2].wait()
        fire_chunk(cc, b)
        if cc >= 1:
            p = 1 - b
            drain_chunk(cc - 1, p)
            stores[cc - 1] = pltpu.async_copy(
                rows[p],
                out_hbm.at[pl.ds(ibase + (cc - 1) * _ICHUNK, _ICHUNK)],
                ssems[p])
    last = _NCHUNK - 1
    drain_chunk(last, last % 2)
    stores[last] = pltpu.async_copy(
        rows[last % 2], out_hbm.at[pl.ds(ibase + last * _ICHUNK, _ICHUNK)],
        ssems[last % 2])
    stores[last - 1].wait()
    stores[last].wait()


def kernel(x, table):
    xT = jnp.transpose(x.astype(jnp.int32))
    tT = jnp.transpose(table)
    out, _ = _embed_kernel(xT, tT)
    return out


# speedup vs baseline: 4.4036x; 3.9161x over previous
import functools

import jax
import jax.numpy as jnp
from jax import lax
from jax.experimental import pallas as pl
from jax.experimental.pallas import tpu as pltpu
from jax.experimental.pallas import tpu_sc as plsc

VOCAB = 1000000
DIM = 32
SEQ = 50
BATCH = 16384

_info = plsc.get_sparse_core_info()
_NC, _NS = _info.num_cores, _info.num_subcores
_NW = _NC * _NS
_IPW = BATCH // _NW
_ICHUNK = 32
_NCHUNK = _IPW // _ICHUNK


@functools.partial(
    pl.kernel,
    mesh=plsc.VectorSubcoreMesh(core_axis_name="c", subcore_axis_name="s"),
    out_type=jax.ShapeDtypeStruct((BATCH, SEQ, DIM), jnp.float32),
    scratch_types=[
        pltpu.VMEM((_ICHUNK, SEQ), jnp.int32),
        pltpu.VMEM((_ICHUNK, SEQ), jnp.int32),
        pltpu.VMEM((_ICHUNK, SEQ, DIM), jnp.float32),
        pltpu.VMEM((_ICHUNK, SEQ, DIM), jnp.float32),
        pltpu.SemaphoreType.DMA,
        pltpu.SemaphoreType.DMA,
        pltpu.SemaphoreType.DMA,
        pltpu.SemaphoreType.DMA,
    ],
    compiler_params=pltpu.CompilerParams(use_tc_tiling_on_sc=False),
)
def _gather_kernel(x_hbm, table_hbm, out_hbm, idx0, idx1, rows0, rows1,
                   gsem0, gsem1, ssem0, ssem1):
    wid = lax.axis_index("s") * _NC + lax.axis_index("c")
    ibase = wid * _IPW
    idxs = (idx0, idx1)
    rows = (rows0, rows1)
    gsems = (gsem0, gsem1)
    ssems = (ssem0, ssem1)

    def fire_chunk(c, b):
        i0 = ibase + c * _ICHUNK
        pltpu.sync_copy(x_hbm.at[pl.ds(i0, _ICHUNK)], idxs[b])

        def gather_one(k, carry):
            pltpu.async_copy(table_hbm.at[idxs[b].at[k]], rows[b].at[k],
                             gsems[b])
            return carry

        lax.fori_loop(0, _ICHUNK, gather_one, 0)

    def drain_chunk(c, b):
        i0 = ibase + c * _ICHUNK
        pltpu.make_async_copy(out_hbm.at[pl.ds(i0, _ICHUNK)], rows[b],
                              gsems[b]).wait()

    stores = [None] * _NCHUNK
    for c in range(_NCHUNK):
        b = c % 2
        if c >= 2:
            stores[c - 2].wait()
        fire_chunk(c, b)
        if c >= 1:
            p = 1 - b
            drain_chunk(c - 1, p)
            stores[c - 1] = pltpu.async_copy(
                rows[p],
                out_hbm.at[pl.ds(ibase + (c - 1) * _ICHUNK, _ICHUNK)],
                ssems[p])
    last = _NCHUNK - 1
    drain_chunk(last, last % 2)
    stores[last] = pltpu.async_copy(
        rows[last % 2], out_hbm.at[pl.ds(ibase + last * _ICHUNK, _ICHUNK)],
        ssems[last % 2])
    stores[last - 1].wait()
    stores[last].wait()


def kernel(x, table):
    return _gather_kernel(x.astype(jnp.int32), table)
